# Initial kernel scaffold; baseline (speedup 1.0000x reference)
import functools

import jax
import jax.numpy as jnp
from jax import lax
from jax.experimental import pallas as pl
from jax.experimental.pallas import tpu as pltpu
from jax.experimental.pallas import tpu_sc as plsc

_B = 16384
_N = 26
_D = 128
_VOCAB_PER_FIELD = 1000
_BN = _B * _N
_NW = 32
_BPW = _BN // _NW
_C = 512
_KIDX = _C // 128
_NCHUNK = _BPW // _C

_mesh = plsc.VectorSubcoreMesh(core_axis_name="c", subcore_axis_name="s")


@functools.partial(
    pl.kernel,
    mesh=_mesh,
    out_type=jax.ShapeDtypeStruct((_BN, _D), jnp.float32),
    scratch_types=[
        pltpu.VMEM((_KIDX, 128), jnp.int32),
        pltpu.VMEM((_KIDX, 128), jnp.int32),
        pltpu.VMEM((_C, _D), jnp.float32),
        pltpu.SemaphoreType.DMA,
    ],
)
def _emb_lookup(x_hbm, off_hbm, table_hbm, out_hbm, idx_v, off_v, rows_v, sem):
    wid = lax.axis_index("s") * 2 + lax.axis_index("c")
    base = wid * _BPW

    def body(c, _):
        start = base + c * _C
        r0 = start // 128
        pltpu.sync_copy(x_hbm.at[pl.ds(r0, _KIDX)], idx_v)
        pltpu.sync_copy(off_hbm.at[pl.ds(r0, _KIDX)], off_v)
        for j in range(_KIDX):
            for t in range(8):
                sl = pl.ds(t * 16, 16)
                idx_v[j, sl] = idx_v[j, sl] + off_v[j, sl]
        copies = [
            pltpu.async_copy(
                table_hbm.at[idx_v.at[j]],
                rows_v.at[pl.ds(j * 128, 128)],
                sem,
            )
            for j in range(_KIDX)
        ]
        for cp in copies:
            cp.wait()
        pltpu.sync_copy(rows_v, out_hbm.at[pl.ds(start, _C)])
        return ()

    lax.fori_loop(0, _NCHUNK, body, ())


def kernel(x, table):
    x2 = x.astype(jnp.int32).reshape(_BN // 128, 128)
    off2 = jnp.tile(
        jnp.arange(_N, dtype=jnp.int32) * _VOCAB_PER_FIELD, _B
    ).reshape(_BN // 128, 128)
    out = _emb_lookup(x2, off2, table)
    return out.reshape(_B, _N, _D)


# speedup vs baseline: 3.0926x; 3.0926x over previous
import functools

import jax
import jax.numpy as jnp
from jax import lax
from jax.experimental import pallas as pl
from jax.experimental.pallas import tpu as pltpu
from jax.experimental.pallas import tpu_sc as plsc

_B = 16384
_N = 26
_D = 128
_VOCAB_PER_FIELD = 1000
_BN = _B * _N
_NW = 32
_BPW = _BN // _NW
_C = 512
_KIDX = _C // 128
_NCHUNK = _BPW // _C

_mesh = plsc.VectorSubcoreMesh(core_axis_name="c", subcore_axis_name="s")


@functools.partial(
    pl.kernel,
    mesh=_mesh,
    out_type=jax.ShapeDtypeStruct((_BN, _D), jnp.float32),
    scratch_types=[
        pltpu.VMEM((_C,), jnp.int32),
        pltpu.VMEM((_C,), jnp.int32),
        pltpu.VMEM((_KIDX, 128), jnp.int32),
        pltpu.VMEM((_C, _D), jnp.float32),
        pltpu.SemaphoreType.DMA,
    ],
)
def _emb_lookup(x_hbm, off_hbm, table_hbm, out_hbm, xv, off_v, idx_v, rows_v,
                sem):
    wid = lax.axis_index("s") * 2 + lax.axis_index("c")
    base = wid * _BPW

    def body(c, _):
        start = base + c * _C
        pltpu.sync_copy(x_hbm.at[pl.ds(start, _C)], xv)
        pltpu.sync_copy(off_hbm.at[pl.ds(start, _C)], off_v)
        for j in range(_KIDX):
            for t in range(8):
                sl = pl.ds(j * 128 + t * 16, 16)
                idx_v[j, pl.ds(t * 16, 16)] = xv[sl] + off_v[sl]
        copies = [
            pltpu.async_copy(
                table_hbm.at[idx_v.at[j]],
                rows_v.at[pl.ds(j * 128, 128)],
                sem,
            )
            for j in range(_KIDX)
        ]
        for cp in copies:
            cp.wait()
        pltpu.sync_copy(rows_v, out_hbm.at[pl.ds(start, _C)])
        return ()

    lax.fori_loop(0, _NCHUNK, body, ())


def kernel(x, table):
    xf = x.astype(jnp.int32).reshape(_BN)
    off = jnp.tile(jnp.arange(_N, dtype=jnp.int32) * _VOCAB_PER_FIELD, _B)
    out = _emb_lookup(xf, off, table)
    return out.reshape(_B, _N, _D)


# speedup vs baseline: 3.2419x; 1.0483x over previous
import functools

import jax
import jax.numpy as jnp
from jax import lax
from jax.experimental import pallas as pl
from jax.experimental.pallas import tpu as pltpu
from jax.experimental.pallas import tpu_sc as plsc

_B = 16384
_N = 26
_D = 128
_VOCAB_PER_FIELD = 1000
_BN = _B * _N
_NW = 32
_BPW = _BN // _NW
_C = 256
_KIDX = _C // 128
_NCHUNK = _BPW // _C

_mesh = plsc.VectorSubcoreMesh(core_axis_name="c", subcore_axis_name="s")


@functools.partial(
    pl.kernel,
    mesh=_mesh,
    out_type=jax.ShapeDtypeStruct((_BN, _D), jnp.float32),
    scratch_types=[
        pltpu.VMEM((_BPW,), jnp.int32),
        pltpu.VMEM((_BPW,), jnp.int32),
        pltpu.VMEM((_BPW,), jnp.int32),
        pltpu.VMEM((_C, _D), jnp.float32),
        pltpu.VMEM((_C, _D), jnp.float32),
        pltpu.SemaphoreType.DMA,
        pltpu.SemaphoreType.DMA,
        pltpu.SemaphoreType.DMA,
    ],
)
def _emb_lookup(x_hbm, off_hbm, table_hbm, out_hbm,
                xv, off_v, idx_v, rows0, rows1, gsem, osem0, osem1):
    wid = lax.axis_index("s") * 2 + lax.axis_index("c")
    base = wid * _BPW

    pltpu.sync_copy(x_hbm.at[pl.ds(base, _BPW)], xv)
    pltpu.sync_copy(off_hbm.at[pl.ds(base, _BPW)], off_v)

    def add_body(i, _):
        sl = pl.ds(i * 16, 16)
        idx_v[sl] = xv[sl] + off_v[sl]
        return ()

    lax.fori_loop(0, _BPW // 16, add_body, ())

    def chunk(c, rows, osem):
        @pl.when(c >= 2)
        def _():
            pltpu.make_async_copy(
                rows, out_hbm.at[pl.ds(base + (c - 2) * _C, _C)], osem
            ).wait()

        copies = [
            pltpu.async_copy(
                table_hbm.at[idx_v.at[pl.ds(c * _C + k * 128, 128)]],
                rows.at[pl.ds(k * 128, 128)],
                gsem,
            )
            for k in range(_KIDX)
        ]
        for cp in copies:
            cp.wait()
        pltpu.async_copy(rows, out_hbm.at[pl.ds(base + c * _C, _C)], osem)

    def body(i, _):
        chunk(2 * i, rows0, osem0)
        chunk(2 * i + 1, rows1, osem1)
        return ()

    lax.fori_loop(0, _NCHUNK // 2, body, ())

    pltpu.make_async_copy(
        rows0, out_hbm.at[pl.ds(base + (_NCHUNK - 2) * _C, _C)], osem0
    ).wait()
    pltpu.make_async_copy(
        rows1, out_hbm.at[pl.ds(base + (_NCHUNK - 1) * _C, _C)], osem1
    ).wait()


def kernel(x, table):
    xf = x.astype(jnp.int32).reshape(_BN)
    off = jnp.tile(jnp.arange(_N, dtype=jnp.int32) * _VOCAB_PER_FIELD, _B)
    out = _emb_lookup(xf, off, table)
    return out.reshape(_B, _N, _D)


# speedup vs baseline: 5.1800x; 1.5978x over previous
import functools

import jax
import jax.numpy as jnp
from jax import lax
from jax.experimental import pallas as pl
from jax.experimental.pallas import tpu as pltpu
from jax.experimental.pallas import tpu_sc as plsc

_B = 16384
_N = 26
_D = 128
_VOCAB_PER_FIELD = 1000
_BN = _B * _N
_NW = 32
_BROWS = _B // _NW
_BPW = _BN // _NW
_CB = 8
_C = _CB * _N
_NCHUNK = _BROWS // _CB

_mesh = plsc.VectorSubcoreMesh(core_axis_name="c", subcore_axis_name="s")


@functools.partial(
    pl.kernel,
    mesh=_mesh,
    out_type=jax.ShapeDtypeStruct((_B, _N, _D), jnp.float32),
    scratch_types=[
        pltpu.VMEM((_BPW,), jnp.int32),
        pltpu.VMEM((_BPW,), jnp.int32),
        pltpu.VMEM((_BPW,), jnp.int32),
        pltpu.VMEM((_C, _D), jnp.float32),
        pltpu.VMEM((_C, _D), jnp.float32),
        pltpu.SemaphoreType.DMA,
        pltpu.SemaphoreType.DMA,
        pltpu.SemaphoreType.DMA,
    ],
)
def _emb_lookup(x_hbm, off_hbm, table_hbm, out_hbm,
                xv, off_v, idx_v, rows0, rows1, gsem, osem0, osem1):
    wid = lax.axis_index("s") * 2 + lax.axis_index("c")
    base = wid * _BPW
    brow0 = wid * _BROWS

    pltpu.sync_copy(x_hbm.at[pl.ds(base, _BPW)], xv)
    pltpu.sync_copy(off_hbm.at[pl.ds(base, _BPW)], off_v)

    def add_body(i, _):
        sl = pl.ds(i * 16, 16)
        idx_v[sl] = xv[sl] + off_v[sl]
        return ()

    lax.fori_loop(0, _BPW // 16, add_body, ())

    def drain_out(rows, osem):
        for _ in range(_CB):
            pltpu.make_async_copy(
                rows.at[pl.ds(0, _N)], out_hbm.at[brow0], osem
            ).wait()

    def chunk(c, rows, osem):
        @pl.when(c >= 2)
        def _():
            drain_out(rows, osem)

        copies = [
            pltpu.async_copy(
                table_hbm.at[idx_v.at[pl.ds(c * _C, 128)]],
                rows.at[pl.ds(0, 128)],
                gsem,
            ),
            pltpu.async_copy(
                table_hbm.at[idx_v.at[pl.ds(c * _C + 128, _C - 128)]],
                rows.at[pl.ds(128, _C - 128)],
                gsem,
            ),
        ]
        for cp in copies:
            cp.wait()
        for r in range(_CB):
            pltpu.async_copy(
                rows.at[pl.ds(r * _N, _N)],
                out_hbm.at[brow0 + c * _CB + r],
                osem,
            )

    def body(i, _):
        chunk(2 * i, rows0, osem0)
        chunk(2 * i + 1, rows1, osem1)
        return ()

    lax.fori_loop(0, _NCHUNK // 2, body, ())

    drain_out(rows0, osem0)
    drain_out(rows1, osem1)


def kernel(x, table):
    xf = x.astype(jnp.int32).reshape(_BN)
    off = jnp.tile(jnp.arange(_N, dtype=jnp.int32) * _VOCAB_PER_FIELD, _B)
    return _emb_lookup(xf, off, table)


# speedup vs baseline: 11.1012x; 2.1431x over previous
import functools

import jax
import jax.numpy as jnp
from jax import lax
from jax.experimental import pallas as pl
from jax.experimental.pallas import tpu as pltpu
from jax.experimental.pallas import tpu_sc as plsc

_B = 16384
_N = 26
_D = 128
_VOCAB_PER_FIELD = 1000
_LOG2_B = 14
_BN = _B * _N
_NW = 32
_BPW = _BN // _NW
_C = 256
_KIDX = _C // 128
_NCHUNK = _BPW // _C

_mesh = plsc.VectorSubcoreMesh(core_axis_name="c", subcore_axis_name="s")


@functools.partial(
    pl.kernel,
    mesh=_mesh,
    out_type=jax.ShapeDtypeStruct((_BN, _D), jnp.float32),
    scratch_types=[
        pltpu.VMEM((_BPW,), jnp.int32),
        pltpu.VMEM((_BPW,), jnp.int32),
        pltpu.VMEM((_C, _D), jnp.float32),
        pltpu.VMEM((_C, _D), jnp.float32),
        pltpu.SemaphoreType.DMA,
        pltpu.SemaphoreType.DMA,
        pltpu.SemaphoreType.DMA,
    ],
)
def _emb_lookup(x_hbm, table_hbm, out_hbm,
                xv, idx_v, rows0, rows1, gsem, osem0, osem1):
    wid = lax.axis_index("s") * 2 + lax.axis_index("c")
    base = wid * _BPW

    pltpu.sync_copy(x_hbm.at[pl.ds(base, _BPW)], xv)

    def add_body(i, _):
        sl = pl.ds(i * 16, 16)
        pos = lax.iota(jnp.int32, 16) + (base + i * 16)
        off = lax.shift_right_logical(pos, _LOG2_B) * _VOCAB_PER_FIELD
        idx_v[sl] = xv[sl] + off
        return ()

    lax.fori_loop(0, _BPW // 16, add_body, ())

    def chunk(c, rows, osem):
        @pl.when(c >= 2)
        def _():
            pltpu.make_async_copy(
                rows, out_hbm.at[pl.ds(base + (c - 2) * _C, _C)], osem
            ).wait()

        copies = [
            pltpu.async_copy(
                table_hbm.at[idx_v.at[pl.ds(c * _C + k * 128, 128)]],
                rows.at[pl.ds(k * 128, 128)],
                gsem,
            )
            for k in range(_KIDX)
        ]
        for cp in copies:
            cp.wait()
        pltpu.async_copy(rows, out_hbm.at[pl.ds(base + c * _C, _C)], osem)

    def body(i, _):
        chunk(2 * i, rows0, osem0)
        chunk(2 * i + 1, rows1, osem1)
        return ()

    lax.fori_loop(0, _NCHUNK // 2, body, ())

    pltpu.make_async_copy(
        rows0, out_hbm.at[pl.ds(base + (_NCHUNK - 2) * _C, _C)], osem0
    ).wait()
    pltpu.make_async_copy(
        rows1, out_hbm.at[pl.ds(base + (_NCHUNK - 1) * _C, _C)], osem1
    ).wait()


def kernel(x, table):
    xf = jnp.transpose(x.astype(jnp.int32)).reshape(_BN)
    out = _emb_lookup(xf, table)
    return jnp.swapaxes(out.reshape(_N, _B, _D), 0, 1)


# speedup vs baseline: 11.3146x; 1.0192x over previous
import functools

import jax
import jax.numpy as jnp
from jax import lax
from jax.experimental import pallas as pl
from jax.experimental.pallas import tpu as pltpu
from jax.experimental.pallas import tpu_sc as plsc

_B = 16384
_N = 26
_D = 128
_VOCAB_PER_FIELD = 1000
_LOG2_B = 14
_BN = _B * _N
_NW = 32
_BPW = _BN // _NW
_C = 256
_KIDX = _C // 128
_NCHUNK = _BPW // _C
_NBUF = 3

_mesh = plsc.VectorSubcoreMesh(core_axis_name="c", subcore_axis_name="s")


@functools.partial(
    pl.kernel,
    mesh=_mesh,
    out_type=jax.ShapeDtypeStruct((_BN, _D), jnp.float32),
    scratch_types=[
        pltpu.VMEM((_BPW,), jnp.int32),
        pltpu.VMEM((_BPW,), jnp.int32),
        pltpu.VMEM((_C, _D), jnp.float32),
        pltpu.VMEM((_C, _D), jnp.float32),
        pltpu.VMEM((_C, _D), jnp.float32),
        pltpu.SemaphoreType.DMA,
        pltpu.SemaphoreType.DMA,
        pltpu.SemaphoreType.DMA,
        pltpu.SemaphoreType.DMA,
    ],
)
def _emb_lookup(x_hbm, table_hbm, out_hbm,
                xv, idx_v, rows0, rows1, rows2,
                gsem, osem0, osem1, osem2):
    wid = lax.axis_index("s") * 2 + lax.axis_index("c")
    base = wid * _BPW

    pltpu.sync_copy(x_hbm.at[pl.ds(base, _BPW)], xv)

    def compute_idx(c):
        def add_body(i, _):
            sl = pl.ds(c * _C + i * 16, 16)
            pos = lax.iota(jnp.int32, 16) + (base + c * _C + i * 16)
            off = lax.shift_right_logical(pos, _LOG2_B) * _VOCAB_PER_FIELD
            idx_v[sl] = xv[sl] + off
            return ()

        lax.fori_loop(0, _C // 16, add_body, ())

    def chunk(c, rows, osem):
        @pl.when(c >= _NBUF)
        def _():
            pltpu.make_async_copy(
                rows, out_hbm.at[pl.ds(base + (c - _NBUF) * _C, _C)], osem
            ).wait()

        copies = [
            pltpu.async_copy(
                table_hbm.at[idx_v.at[pl.ds(c * _C + k * 128, 128)]],
                rows.at[pl.ds(k * 128, 128)],
                gsem,
            )
            for k in range(_KIDX)
        ]

        @pl.when(c + 1 < _NCHUNK)
        def _():
            compute_idx(c + 1)

        for cp in copies:
            cp.wait()
        pltpu.async_copy(rows, out_hbm.at[pl.ds(base + c * _C, _C)], osem)

    compute_idx(0)

    def body(i, _):
        chunk(_NBUF * i, rows0, osem0)
        chunk(_NBUF * i + 1, rows1, osem1)
        chunk(_NBUF * i + 2, rows2, osem2)
        return ()

    lax.fori_loop(0, _NCHUNK // _NBUF, body, ())
    chunk(_NCHUNK - 1, rows0, osem0)

    pltpu.make_async_copy(
        rows1, out_hbm.at[pl.ds(base + (_NCHUNK - 3) * _C, _C)], osem1
    ).wait()
    pltpu.make_async_copy(
        rows2, out_hbm.at[pl.ds(base + (_NCHUNK - 2) * _C, _C)], osem2
    ).wait()
    pltpu.make_async_copy(
        rows0, out_hbm.at[pl.ds(base + (_NCHUNK - 1) * _C, _C)], osem0
    ).wait()


def kernel(x, table):
    xf = jnp.transpose(x.astype(jnp.int32)).reshape(_BN)
    out = _emb_lookup(xf, table)
    return jnp.swapaxes(out.reshape(_N, _B, _D), 0, 1)
